# baseline (speedup 1.0000x reference)
import numpy as np
import jax
import jax.numpy as jnp
from jax.experimental import pallas as pl
from jax.experimental.pallas import tpu as pltpu

D = 256
M = 64
T = 20
NUM_CLASSES = 10
STEADY_T = 6
NSLOT = 5


def _band_coeffs():
    offsets = np.arange(-2, 3)
    wslot = np.zeros((T, M), dtype=np.float64)
    for t in range(T):
        idx = (t + offsets) % M
        delta = idx.astype(np.float32) - np.float32(t)
        logits = (-(delta.astype(np.float32) ** 2) / np.float32(8.0)).astype(np.float32)
        e = np.exp(logits).astype(np.float32)
        w = (e / e.sum(dtype=np.float32)).astype(np.float32)
        wslot[t, idx] = w
    return np.tril(wslot @ wslot.T, -1)


_C = _band_coeffs()
_CS = [np.float32(_C[STEADY_T, STEADY_T - d]) for d in range(1, 5)]


def _layer_norm(ht, gamma, beta):
    mu = jnp.mean(ht, axis=1, keepdims=True)
    var = jnp.mean((ht - mu) * (ht - mu), axis=1, keepdims=True)
    return (ht - mu) * jax.lax.rsqrt(var + 1e-5) * gamma + beta


def _recurrence_kernel(x_ref, we_ref, be_ref, wut_ref, bu_ref, gamma_ref,
                       beta_ref, wo_ref, bo_ref, cs_ref, out_ref,
                       hbuf_ref, wband_ref):
    x = x_ref[...]
    we = we_ref[...]
    be = be_ref[...]
    wut = wut_ref[...]
    bu = bu_ref[...]
    gamma = gamma_ref[...]
    beta = beta_ref[...]
    cs = jax.nn.sigmoid(cs_ref[0, 0])

    B = x.shape[0]

    for v in range(4):
        for s in range(4):
            d = ((v - s - 1) % 4) + 1
            coeff = cs * _CS[d - 1] + (np.float32(1.0) if d == 1 else np.float32(0.0))
            wband_ref[v, s * D:(s + 1) * D, :] = coeff * wut
        wband_ref[v, 4 * D:5 * D, :] = wut

    h = jnp.zeros((B, D), jnp.float32)
    hist = []
    for t in range(T):
        inp = jnp.tanh(x[:, t:t + 1] * we + be)
        if t < STEADY_T:
            ctx_terms = []
            for d in range(1, 5):
                u = t - d
                if u >= 0 and _C[t, u] != 0.0:
                    ctx_terms.append(np.float32(_C[t, u]) * hist[u])
            if ctx_terms:
                ctx = ctx_terms[0]
                for term in ctx_terms[1:]:
                    ctx = ctx + term
                pre_in = inp + cs * ctx + h
            else:
                pre_in = inp + h
            pre = jax.lax.dot_general(
                pre_in, wut, (((1,), (0,)), ((), ())),
                preferred_element_type=jnp.float32) + bu
        else:
            hbuf_ref[:, 4 * D:5 * D] = inp
            v = t % 4
            pre = jax.lax.dot_general(
                hbuf_ref[...], wband_ref[v], (((1,), (0,)), ((), ())),
                preferred_element_type=jnp.float32) + bu
        ht = _layer_norm(jnp.tanh(pre), gamma, beta)
        h = ht
        if t < STEADY_T:
            hist.append(ht)
        if t >= 2 and t < T - 1:
            hbuf_ref[:, (t % 4) * D:(t % 4 + 1) * D] = ht

    wo = wo_ref[...]
    bo = bo_ref[...]
    out_ref[...] = jax.lax.dot_general(
        h, wo, (((1,), (1,)), ((), ())),
        preferred_element_type=jnp.float32) + bo


def kernel(x, W_embed, b_embed, W_update, b_update, gamma, beta, W_out,
           b_out, ctx_strength):
    B = x.shape[0]
    x2 = x.reshape(B, T)
    we = W_embed.reshape(1, D)
    be = b_embed.reshape(1, D)
    bu = b_update.reshape(1, D)
    g = gamma.reshape(1, D)
    bt = beta.reshape(1, D)
    bo = b_out.reshape(1, NUM_CLASSES)
    cs = jnp.reshape(ctx_strength, (1, 1))

    return pl.pallas_call(
        _recurrence_kernel,
        out_shape=jax.ShapeDtypeStruct((B, NUM_CLASSES), jnp.float32),
        scratch_shapes=[
            pltpu.VMEM((B, NSLOT * D), jnp.float32),
            pltpu.VMEM((4, NSLOT * D, D), jnp.float32),
        ],
    )(x2, we, be, W_update.T, bu, g, bt, W_out, bo, cs)


# speedup vs baseline: 1.3891x; 1.3891x over previous
import numpy as np
import jax
import jax.numpy as jnp
from jax.experimental import pallas as pl

D = 256
M = 64
T = 20
NUM_CLASSES = 10


def _band_coeffs():
    offsets = np.arange(-2, 3)
    wslot = np.zeros((T, M), dtype=np.float64)
    for t in range(T):
        idx = (t + offsets) % M
        delta = idx.astype(np.float32) - np.float32(t)
        logits = (-(delta.astype(np.float32) ** 2) / np.float32(8.0)).astype(np.float32)
        e = np.exp(logits).astype(np.float32)
        w = (e / e.sum(dtype=np.float32)).astype(np.float32)
        wslot[t, idx] = w
    return np.tril(wslot @ wslot.T, -1)


_C = _band_coeffs()


def _recurrence_kernel(x_ref, we_ref, be_ref, wu_ref, bu_ref, gamma_ref,
                       beta_ref, wo_ref, bo_ref, cs_ref, out_ref):
    x = x_ref[...]
    we = we_ref[...]
    be = be_ref[...]
    wu = wu_ref[...]
    bu = bu_ref[...]
    gamma = gamma_ref[...]
    beta = beta_ref[...]
    cs = jax.nn.sigmoid(cs_ref[0, 0])

    B = x.shape[0]
    h = jnp.zeros((B, D), jnp.float32)
    hist = []
    for t in range(T):
        inp = jnp.tanh(x[:, t:t + 1] * we + be)
        pre_in = inp
        for d in range(1, 5):
            u = t - d
            if u < 0:
                continue
            coeff = cs * np.float32(_C[t, u]) if _C[t, u] != 0.0 else None
            if d == 1:
                coeff = coeff + np.float32(1.0) if coeff is not None else None
                pre_in = pre_in + (hist[u] if coeff is None else coeff * hist[u])
            elif coeff is not None:
                pre_in = pre_in + coeff * hist[u]
        pre = jax.lax.dot_general(
            pre_in, wu, (((1,), (1,)), ((), ())),
            preferred_element_type=jnp.float32) + bu
        ht = jnp.tanh(pre)
        mu = jnp.mean(ht, axis=1, keepdims=True)
        var = jnp.mean((ht - mu) * (ht - mu), axis=1, keepdims=True)
        ht = (ht - mu) * jax.lax.rsqrt(var + 1e-5) * gamma + beta
        h = ht
        hist.append(ht)

    wo = wo_ref[...]
    bo = bo_ref[...]
    out_ref[...] = jax.lax.dot_general(
        h, wo, (((1,), (1,)), ((), ())),
        preferred_element_type=jnp.float32) + bo


def kernel(x, W_embed, b_embed, W_update, b_update, gamma, beta, W_out,
           b_out, ctx_strength):
    B = x.shape[0]
    x2 = x.reshape(B, T)
    we = W_embed.reshape(1, D)
    be = b_embed.reshape(1, D)
    bu = b_update.reshape(1, D)
    g = gamma.reshape(1, D)
    bt = beta.reshape(1, D)
    bo = b_out.reshape(1, NUM_CLASSES)
    cs = jnp.reshape(ctx_strength, (1, 1))

    return pl.pallas_call(
        _recurrence_kernel,
        out_shape=jax.ShapeDtypeStruct((B, NUM_CLASSES), jnp.float32),
    )(x2, we, be, W_update, bu, g, bt, W_out, bo, cs)
